# baseline (speedup 1.0000x reference)
import functools

import jax
import jax.numpy as jnp
from jax import lax
from jax.experimental import pallas as pl
from jax.experimental.pallas import tpu as pltpu
from jax.experimental.pallas import tpu_sc as plsc

_NC = 2
_NS = 16
_NW = _NC * _NS
_L = 16


def _nn3_kernel(n2, blk, x1_ref, x2t_ref, idx_ref, w_ref):
    b = pl.program_id(0)
    x1 = x1_ref[0]
    x2t = x2t_ref[0]
    d2 = None
    for c in range(3):
        diff = x1[:, c:c + 1] - x2t[c:c + 1, :]
        d2 = diff * diff if d2 is None else d2 + diff * diff
    j = lax.broadcasted_iota(jnp.int32, d2.shape, 1)
    idxs, invs = [], []
    for k in range(3):
        m = jnp.min(d2, axis=1, keepdims=True)
        ik = jnp.min(jnp.where(d2 == m, j, n2), axis=1, keepdims=True)
        idxs.append(ik)
        invs.append(1.0 / jnp.maximum(m, 1e-10))
        if k < 2:
            d2 = jnp.where(j == ik, jnp.inf, d2)
    norm = invs[0] + invs[1] + invs[2]
    idx_ref[0] = jnp.concatenate([ik + b * n2 for ik in idxs], axis=1)
    w_ref[0] = jnp.concatenate(
        [jnp.broadcast_to(inv / norm, (blk, _L)) for inv in invs], axis=1)


def _mlp_kernel(it_ref, p1_ref, w0a_ref, w0b_ref, b0_ref, w1_ref, b1_ref,
                o_ref):
    h = jnp.dot(it_ref[...], w0a_ref[...], preferred_element_type=jnp.float32)
    h = h + jnp.dot(p1_ref[...], w0b_ref[...],
                    preferred_element_type=jnp.float32)
    h = jnp.maximum(h + b0_ref[...], 0.0)
    o = jnp.dot(h, w1_ref[...], preferred_element_type=jnp.float32)
    o_ref[...] = jnp.maximum(o + b1_ref[...], 0.0)


def kernel(xyz1, xyz2, points1, points2, W0, b0, W1, b1):
    B, N1, _ = xyz1.shape
    N2 = xyz2.shape[1]
    C1 = points1.shape[2]
    C2 = points2.shape[2]
    H = W0.shape[1]
    H2 = W1.shape[1]
    Q = N1

    BLK = 512
    nn3 = pl.pallas_call(
        functools.partial(_nn3_kernel, N2, BLK),
        grid=(1, N1 // BLK),
        in_specs=[
            pl.BlockSpec((1, BLK, 3), lambda b, n: (b, n, 0)),
            pl.BlockSpec((1, 3, N2), lambda b, n: (b, 0, 0)),
        ],
        out_specs=[
            pl.BlockSpec((1, BLK, 3), lambda b, n: (b, n, 0)),
            pl.BlockSpec((1, BLK, 3 * _L), lambda b, n: (b, n, 0)),
        ],
        out_shape=[
            jax.ShapeDtypeStruct((1, N1, 3), jnp.int32),
            jax.ShapeDtypeStruct((1, N1, 3 * _L), jnp.float32),
        ],
    )

    QPW = Q // _NW
    CH = 32
    NCH = QPW // CH
    nf = C2 // _L
    mesh = plsc.VectorSubcoreMesh(core_axis_name="c", subcore_axis_name="s")

    @functools.partial(
        pl.kernel,
        mesh=mesh,
        out_type=jax.ShapeDtypeStruct((Q, C2), jnp.float32),
        scratch_types=[
            pltpu.VMEM((QPW * 3,), jnp.int32),
            pltpu.VMEM((CH * 3 * _L,), jnp.float32),
            pltpu.VMEM((CH * 3 * _L,), jnp.float32),
            pltpu.VMEM((CH * 3, C2), jnp.float32),
            pltpu.VMEM((CH * 3, C2), jnp.float32),
            pltpu.VMEM((CH, C2), jnp.float32),
            pltpu.VMEM((CH, C2), jnp.float32),
            pltpu.SemaphoreType.DMA,
            pltpu.SemaphoreType.DMA,
            pltpu.SemaphoreType.DMA,
            pltpu.SemaphoreType.DMA,
            pltpu.SemaphoreType.DMA,
            pltpu.SemaphoreType.DMA,
        ],
    )
    def sc_interp(p2_hbm, idx_hbm, w_hbm, out_hbm, idx_v, w_b0, w_b1, r_b0,
                  r_b1, o_b0, o_b1, sg0, sg1, sw0, sw1, so0, so1):
        wid = lax.axis_index("s") * _NC + lax.axis_index("c")
        qw = wid * QPW
        w_b, r_b, o_b = [w_b0, w_b1], [r_b0, r_b1], [o_b0, o_b1]
        sg, sw, so = [sg0, sg1], [sw0, sw1], [so0, so1]
        gd, wd, od = [None, None], [None, None], [None, None]

        pltpu.sync_copy(idx_hbm.at[pl.ds(qw * 3, QPW * 3)], idx_v)

        def start(ci):
            buf = ci % 2
            gd[buf] = pltpu.async_copy(
                p2_hbm.at[idx_v.at[pl.ds(ci * CH * 3, CH * 3)]], r_b[buf],
                sg[buf])
            wd[buf] = pltpu.async_copy(
                w_hbm.at[pl.ds((qw + ci * CH) * 3 * _L, CH * 3 * _L)],
                w_b[buf], sw[buf])

        start(0)
        for ci in range(NCH):
            buf = ci % 2
            if ci + 1 < NCH:
                start(ci + 1)
            gd[buf].wait()
            wd[buf].wait()
            if od[buf] is not None:
                od[buf].wait()
            rows, wv, ov = r_b[buf], w_b[buf], o_b[buf]

            def body(i, _):
                for u in range(2):
                    q = 2 * i + u
                    base = q * 3 * _L
                    w0v = wv[pl.ds(base, _L)]
                    w1v = wv[pl.ds(base + _L, _L)]
                    w2v = wv[pl.ds(base + 2 * _L, _L)]
                    for f in range(nf):
                        sl = pl.ds(f * _L, _L)
                        acc = w0v * rows[3 * q, sl]
                        acc = acc + w1v * rows[3 * q + 1, sl]
                        acc = acc + w2v * rows[3 * q + 2, sl]
                        ov[q, sl] = acc
                return 0

            lax.fori_loop(0, CH // 2, body, 0)
            od[buf] = pltpu.async_copy(
                ov, out_hbm.at[pl.ds(qw + ci * CH, CH)], so[buf])
        od[0].wait()
        od[1].wait()

    MB = 1024
    mlp = pl.pallas_call(
        _mlp_kernel,
        grid=(Q // MB,),
        in_specs=[
            pl.BlockSpec((MB, C2), lambda r: (r, 0)),
            pl.BlockSpec((MB, C1), lambda r: (r, 0)),
            pl.BlockSpec((C2, H), lambda r: (0, 0)),
            pl.BlockSpec((C1, H), lambda r: (0, 0)),
            pl.BlockSpec((1, H), lambda r: (0, 0)),
            pl.BlockSpec((H, H2), lambda r: (0, 0)),
            pl.BlockSpec((1, H2), lambda r: (0, 0)),
        ],
        out_specs=pl.BlockSpec((MB, H2), lambda r: (r, 0)),
        out_shape=jax.ShapeDtypeStruct((Q, H2), jnp.float32),
    )

    xyz2t = xyz2.transpose(0, 2, 1)
    W0a, W0b = W0[:C2], W0[C2:]
    b0r, b1r = b0.reshape(1, H), b1.reshape(1, H2)
    interps = []
    for b in range(B):
        idx3, w3 = nn3(xyz1[b:b + 1], xyz2t[b:b + 1])
        interps.append(sc_interp(points2[b], idx3.reshape(Q * 3),
                                 w3.reshape(Q * 3 * _L)))
    outs = [mlp(interp, points1[b], W0a, W0b, b0r, W1, b1r)
            for b, interp in enumerate(interps)]
    return jnp.stack(outs)


# speedup vs baseline: 1.2562x; 1.2562x over previous
import functools

import jax
import jax.numpy as jnp
from jax import lax
from jax.experimental import pallas as pl
from jax.experimental.pallas import tpu as pltpu
from jax.experimental.pallas import tpu_sc as plsc

_NC = 2
_NS = 16
_NW = _NC * _NS
_L = 16


def _nn3_kernel(n2, blk, boff, x1_ref, x2t_ref, u_ref, *i_refs):
    x1 = x1_ref[0]
    x2t = x2t_ref[0]
    d2 = None
    for c in range(3):
        diff = x1[:, c:c + 1] - x2t[c:c + 1, :]
        d2 = diff * diff if d2 is None else d2 + diff * diff
    jf = lax.broadcasted_iota(jnp.int32, d2.shape, 1).astype(jnp.float32)
    idxs, invs = [], []
    for k in range(3):
        m = jnp.min(d2, axis=1, keepdims=True)
        ikf = jnp.min(jnp.where(d2 == m, jf, float(n2)), axis=1,
                      keepdims=True)
        idxs.append(ikf + float(boff))
        invs.append(1.0 / jnp.maximum(m, 1e-10))
        if k < 2:
            d2 = jnp.where(jf == ikf, jnp.inf, d2)
    norm = invs[0] + invs[1] + invs[2]
    u_ref[...] = jnp.concatenate(
        [jnp.broadcast_to(inv / norm, (blk, _L)) for inv in invs]
        + [jnp.zeros((blk, 128 - 3 * _L), jnp.float32)], axis=1)
    for k in range(3):
        i_refs[k][...] = idxs[k].astype(jnp.int32).reshape(blk)


def _mlp_kernel(it_ref, p1_ref, w0a_ref, w0b_ref, b0_ref, w1_ref, b1_ref,
                o_ref):
    h = jnp.dot(it_ref[...], w0a_ref[...], preferred_element_type=jnp.float32)
    h = h + jnp.dot(p1_ref[...], w0b_ref[...],
                    preferred_element_type=jnp.float32)
    h = jnp.maximum(h + b0_ref[...], 0.0)
    o = jnp.dot(h, w1_ref[...], preferred_element_type=jnp.float32)
    o_ref[...] = jnp.maximum(o + b1_ref[...], 0.0)


def kernel(xyz1, xyz2, points1, points2, W0, b0, W1, b1):
    B, N1, _ = xyz1.shape
    N2 = xyz2.shape[1]
    C1 = points1.shape[2]
    C2 = points2.shape[2]
    H = W0.shape[1]
    H2 = W1.shape[1]
    Q = N1

    BLK = 1024
    QPW = Q // _NW
    CH = 32
    NCH = QPW // CH
    NV = CH * 3 // _L
    nf = C2 // _L
    mesh = plsc.VectorSubcoreMesh(core_axis_name="c", subcore_axis_name="s")

    def make_nn3(b):
        return pl.pallas_call(
            functools.partial(_nn3_kernel, N2, BLK, b * N2),
            grid=(N1 // BLK,),
            in_specs=[
                pl.BlockSpec((1, BLK, 3), lambda n, b=b: (b, n, 0)),
                pl.BlockSpec((1, 3, N2), lambda n, b=b: (b, 0, 0)),
            ],
            out_specs=[pl.BlockSpec((BLK, 128), lambda n: (n, 0))]
            + [pl.BlockSpec((BLK,), lambda n: (n,)) for _ in range(3)],
            out_shape=[jax.ShapeDtypeStruct((N1, 128), jnp.float32)]
            + [jax.ShapeDtypeStruct((N1,), jnp.int32) for _ in range(3)],
        )

    @functools.partial(
        pl.kernel,
        mesh=mesh,
        out_type=jax.ShapeDtypeStruct((Q, C2), jnp.float32),
        scratch_types=[
            pltpu.VMEM((3, CH), jnp.int32),
            pltpu.VMEM((3, CH), jnp.int32),
            pltpu.VMEM((CH * 128,), jnp.float32),
            pltpu.VMEM((CH * 128,), jnp.float32),
            pltpu.VMEM((3, CH, C2), jnp.float32),
            pltpu.VMEM((3, CH, C2), jnp.float32),
            pltpu.VMEM((CH, C2), jnp.float32),
            pltpu.VMEM((CH, C2), jnp.float32),
            pltpu.SemaphoreType.DMA,
            pltpu.SemaphoreType.DMA,
            pltpu.SemaphoreType.DMA,
            pltpu.SemaphoreType.DMA,
            pltpu.SemaphoreType.DMA,
            pltpu.SemaphoreType.DMA,
            pltpu.SemaphoreType.DMA,
            pltpu.SemaphoreType.DMA,
        ],
    )
    def sc_interp(p2_hbm, u_hbm, i0_hbm, i1_hbm, i2_hbm, out_hbm, i_b0,
                  i_b1, u_b0, u_b1, r_b0, r_b1, o_b0, o_b1, si0, si1, su0,
                  su1, sg0, sg1, so0, so1):
        wid = lax.axis_index("s") * _NC + lax.axis_index("c")
        qw = wid * QPW
        i_hbm = [i0_hbm, i1_hbm, i2_hbm]
        i_b, u_b, r_b, o_b = [i_b0, i_b1], [u_b0, u_b1], [r_b0, r_b1], \
            [o_b0, o_b1]
        si, su, sg, so = [si0, si1], [su0, su1], [sg0, sg1], [so0, so1]
        id_, ud, gd, od = [None, None], [None, None], [None, None], \
            [None, None]

        def start_u(ci):
            buf = ci % 2
            sl = pl.ds(qw + ci * CH, CH)
            ud[buf] = pltpu.async_copy(
                u_hbm.at[pl.ds((qw + ci * CH) * 128, CH * 128)], u_b[buf],
                su[buf])
            for k in range(3):
                d = pltpu.async_copy(i_hbm[k].at[sl], i_b[buf].at[k],
                                     si[buf])
            id_[buf] = d

        def start_gather(ci):
            buf = ci % 2
            for k in range(3):
                d = pltpu.async_copy(p2_hbm.at[i_b[buf].at[k]],
                                     r_b[buf].at[k], sg[buf])
            gd[buf] = d

        start_u(0)
        for ci in range(NCH):
            buf = ci % 2
            for k in range(3):
                id_[buf].wait()
            start_gather(ci)
            if ci + 1 < NCH:
                start_u(ci + 1)
            ud[buf].wait()
            for k in range(3):
                gd[buf].wait()
            if od[buf] is not None:
                od[buf].wait()
            rows, uv, ov = r_b[buf], u_b[buf], o_b[buf]

            def body(i, _):
                for u in range(2):
                    q = 2 * i + u
                    w0v = uv[pl.ds(q * 128, _L)]
                    w1v = uv[pl.ds(q * 128 + _L, _L)]
                    w2v = uv[pl.ds(q * 128 + 2 * _L, _L)]
                    for f in range(nf):
                        sl = pl.ds(f * _L, _L)
                        acc = w0v * rows[0, q, sl]
                        acc = acc + w1v * rows[1, q, sl]
                        acc = acc + w2v * rows[2, q, sl]
                        ov[q, sl] = acc
                return 0

            lax.fori_loop(0, CH // 2, body, 0)
            od[buf] = pltpu.async_copy(
                ov, out_hbm.at[pl.ds(qw + ci * CH, CH)], so[buf])
        od[0].wait()
        od[1].wait()

    MB = 1024

    def make_mlp(b):
        return pl.pallas_call(
            _mlp_kernel,
            grid=(Q // MB,),
            in_specs=[
                pl.BlockSpec((MB, C2), lambda r: (r, 0)),
                pl.BlockSpec((MB, C1),
                             lambda r, b=b: (b * (Q // MB) + r, 0)),
                pl.BlockSpec((C2, H), lambda r: (0, 0)),
                pl.BlockSpec((C1, H), lambda r: (0, 0)),
                pl.BlockSpec((1, H), lambda r: (0, 0)),
                pl.BlockSpec((H, H2), lambda r: (0, 0)),
                pl.BlockSpec((1, H2), lambda r: (0, 0)),
            ],
            out_specs=pl.BlockSpec((MB, H2), lambda r: (r, 0)),
            out_shape=jax.ShapeDtypeStruct((Q, H2), jnp.float32),
        )

    xyz2t = xyz2.transpose(0, 2, 1)
    p2flat = points2.reshape(B * N2, C2)
    p1flat = points1.reshape(B * N1, C1)
    W0a, W0b = W0[:C2], W0[C2:]
    b0r, b1r = b0.reshape(1, H), b1.reshape(1, H2)
    interps = []
    for b in range(B):
        u, i0, i1, i2 = make_nn3(b)(xyz1, xyz2t)
        interps.append(sc_interp(p2flat, u.reshape(N1 * 128), i0, i1, i2))
    outs = [make_mlp(b)(interp, p1flat, W0a, W0b, b0r, W1, b1r)
            for b, interp in enumerate(interps)]
    return jnp.stack(outs)
